# baseline (speedup 1.0000x reference)
import numpy as np
import jax
import jax.numpy as jnp
from jax import lax
from jax.experimental import pallas as pl
from jax.experimental.pallas import tpu as pltpu


def _bs_runs(T, B):
    lengths = np.array([T - 32 * i for i in range(B)], dtype=np.int64)
    bs_list = [int((lengths > t).sum()) for t in range(T)]
    runs = []
    for bs in bs_list:
        if runs and runs[-1][0] == bs:
            runs[-1][1] += 1
        else:
            runs.append([bs, 1])
    return [(int(b), int(n)) for b, n in runs]


def _make_kernel(T, B, D, H, total, runs):
    def kern(x_ref, bs_ref, h0_ref, c0_ref, wih_ref, whh_ref, bih_ref,
             bhh_ref, out_ref, hout_ref, cout_ref, xd_ref, gx_ref, outd_ref,
             wihT_ref, whhT_ref):
        col = lax.broadcasted_iota(jnp.int32, (1, 4 * H), 1)
        colscale = jnp.where((col >= 2 * H) & (col < 3 * H),
                             jnp.float32(1.0), jnp.float32(0.5))
        for d in range(2):
            wihT_ref[d] = lax.transpose(wih_ref[d], (1, 0)) * colscale
            whhT_ref[d] = lax.transpose(whh_ref[d], (1, 0)) * colscale
        b0 = (bih_ref[0] + bhh_ref[0]) * colscale
        b1 = (bih_ref[1] + bhh_ref[1]) * colscale
        b_col = (b0, b1)
        t = 0
        off = 0
        for bs, nsteps in runs:
            if bs == B:
                xd_ref[t:t + nsteps] = (
                    x_ref[off:off + nsteps * B, :].reshape(nsteps, B, D))
            else:
                for j in range(nsteps):
                    xd_ref[t + j, 0:bs, :] = (
                        x_ref[off + j * bs:off + (j + 1) * bs, :])
            t += nsteps
            off += bs * nsteps

        LA = 8
        for d, sl in ((0, slice(0, LA)), (1, slice(T - LA, T))):
            gxb = lax.dot_general(
                xd_ref[sl].reshape(LA * B, D), wihT_ref[d],
                (((1,), (0,)), ((), ())),
                preferred_element_type=jnp.float32)
            gx_ref[d, sl] = (gxb + b_col[d]).reshape(LA, B, 4 * H)

        row_id = lax.broadcasted_iota(jnp.int32, (B, 1), 0)
        W0 = whhT_ref[0]
        W1 = whhT_ref[1]

        def prefetch4(t0):
            tpf = jnp.minimum(t0 + LA, T - 4)
            xf = xd_ref[pl.ds(tpf, 4)].reshape(4 * B, D)
            gx_ref[0, pl.ds(tpf, 4)] = (b0 + lax.dot_general(
                xf, wihT_ref[0], (((1,), (0,)), ((), ())),
                preferred_element_type=jnp.float32)).reshape(4, B, 4 * H)
            tpb = jnp.maximum(T - 4 - t0 - LA, 0)
            xb = xd_ref[pl.ds(tpb, 4)].reshape(4 * B, D)
            gx_ref[1, pl.ds(tpb, 4)] = (b1 + lax.dot_general(
                xb, wihT_ref[1], (((1,), (0,)), ((), ())),
                preferred_element_type=jnp.float32)).reshape(4, B, 4 * H)

        def step(t, h, c, W, d):
            gates = gx_ref[d, t] + lax.dot_general(
                h, W, (((1,), (0,)), ((), ())),
                preferred_element_type=jnp.float32)
            a = jnp.tanh(gates)
            ig = 0.5 * a[:, :H] + 0.5
            fg = 0.5 * a[:, H:2 * H] + 0.5
            og = 0.5 * a[:, 3 * H:] + 0.5
            c_new = fg * c + ig * a[:, 2 * H:3 * H]
            h_new = og * jnp.tanh(c_new)
            mask = row_id < bs_ref[t]
            h = jnp.where(mask, h_new, h)
            c = jnp.where(mask, c_new, c)
            outd_ref[t, :, d * H:(d + 1) * H] = h
            return h, c

        def body(i, st):
            hf, cf, hb, cb = st
            t0 = 4 * i
            prefetch4(t0)
            for k in range(4):
                t = t0 + k
                tb = T - 1 - t
                hf, cf = step(t, hf, cf, W0, 0)
                hb, cb = step(tb, hb, cb, W1, 1)
            return (hf, cf, hb, cb)

        hf, cf, hb, cb = lax.fori_loop(
            0, T // 4, body, (h0_ref[0], c0_ref[0], h0_ref[1], c0_ref[1]))
        hout_ref[0] = hf
        cout_ref[0] = cf
        hout_ref[1] = hb
        cout_ref[1] = cb

        t = 0
        off = 0
        for bs, nsteps in runs:
            if bs == B:
                out_ref[off:off + nsteps * B, :] = (
                    outd_ref[t:t + nsteps].reshape(nsteps * B, 2 * H))
            else:
                for j in range(nsteps):
                    out_ref[off + j * bs:off + (j + 1) * bs, :] = (
                        outd_ref[t + j, 0:bs, :])
            t += nsteps
            off += bs * nsteps

    return kern


def kernel(input_data, batch_sizes, h0, c0, W_ih, W_hh, b_ih, b_hh):
    T = batch_sizes.shape[0]
    B = h0.shape[1]
    total, D = input_data.shape
    H = h0.shape[2]
    runs = _bs_runs(T, B)

    kern = _make_kernel(T, B, D, H, total, runs)
    out, h_out, c_out = pl.pallas_call(
        kern,
        out_shape=(
            jax.ShapeDtypeStruct((total, 2 * H), jnp.float32),
            jax.ShapeDtypeStruct((2, B, H), jnp.float32),
            jax.ShapeDtypeStruct((2, B, H), jnp.float32),
        ),
        in_specs=[
            pl.BlockSpec(memory_space=pltpu.VMEM),
            pl.BlockSpec(memory_space=pltpu.SMEM),
            pl.BlockSpec(memory_space=pltpu.VMEM),
            pl.BlockSpec(memory_space=pltpu.VMEM),
            pl.BlockSpec(memory_space=pltpu.VMEM),
            pl.BlockSpec(memory_space=pltpu.VMEM),
            pl.BlockSpec(memory_space=pltpu.VMEM),
            pl.BlockSpec(memory_space=pltpu.VMEM),
        ],
        scratch_shapes=[
            pltpu.VMEM((T, B, D), jnp.float32),
            pltpu.VMEM((2, T, B, 4 * H), jnp.float32),
            pltpu.VMEM((T, B, 2 * H), jnp.float32),
            pltpu.VMEM((2, D, 4 * H), jnp.float32),
            pltpu.VMEM((2, H, 4 * H), jnp.float32),
        ],
    )(
        input_data,
        batch_sizes,
        h0,
        c0,
        W_ih,
        W_hh,
        b_ih.reshape(2, 1, 4 * H),
        b_hh.reshape(2, 1, 4 * H),
    )
    return (out, h_out, c_out)


# speedup vs baseline: 1.1399x; 1.1399x over previous
import numpy as np
import jax
import jax.numpy as jnp
from jax import lax
from jax.experimental import pallas as pl
from jax.experimental.pallas import tpu as pltpu


def _bs_runs(T, B):
    lengths = np.array([T - 32 * i for i in range(B)], dtype=np.int64)
    bs_list = [int((lengths > t).sum()) for t in range(T)]
    runs = []
    for bs in bs_list:
        if runs and runs[-1][0] == bs:
            runs[-1][1] += 1
        else:
            runs.append([bs, 1])
    return [(int(b), int(n)) for b, n in runs]


def _make_kernel(T, B, D, H, total, runs):
    def kern(x_ref, bs_ref, h0_ref, c0_ref, wih_ref, whh_ref, bih_ref,
             bhh_ref, out_ref, hout_ref, cout_ref, xd_ref, gx_ref, outd_ref,
             wihT_ref, whhT_ref):
        col = lax.broadcasted_iota(jnp.int32, (1, 4 * H), 1)
        colscale = jnp.where((col >= 2 * H) & (col < 3 * H),
                             jnp.float32(1.0), jnp.float32(0.5))
        for d in range(2):
            wihT_ref[d] = lax.transpose(wih_ref[d], (1, 0)) * colscale
            whhT_ref[d] = lax.transpose(whh_ref[d], (1, 0)) * colscale
        b0 = (bih_ref[0] + bhh_ref[0]) * colscale
        b1 = (bih_ref[1] + bhh_ref[1]) * colscale
        b_col = (b0, b1)
        t = 0
        off = 0
        for bs, nsteps in runs:
            if bs == B:
                xd_ref[t:t + nsteps] = (
                    x_ref[off:off + nsteps * B, :].reshape(nsteps, B, D))
            else:
                for j in range(nsteps):
                    xd_ref[t + j, 0:bs, :] = (
                        x_ref[off + j * bs:off + (j + 1) * bs, :])
            t += nsteps
            off += bs * nsteps

        LA = 8
        for d, sl in ((0, slice(0, LA)), (1, slice(T - LA, T))):
            gxb = lax.dot_general(
                xd_ref[sl].reshape(LA * B, D), wihT_ref[d],
                (((1,), (0,)), ((), ())),
                preferred_element_type=jnp.float32)
            gx_ref[d, sl] = (gxb + b_col[d]).reshape(LA, B, 4 * H)

        row_id = lax.broadcasted_iota(jnp.int32, (B, 1), 0)
        W0 = whhT_ref[0]
        W1 = whhT_ref[1]

        def prefetch(t):
            tpf = jnp.minimum(t + LA, T - 1)
            gx_ref[0, tpf] = b0 + lax.dot_general(
                xd_ref[tpf], wihT_ref[0], (((1,), (0,)), ((), ())),
                preferred_element_type=jnp.float32)
            tpb = jnp.maximum(T - 1 - t - LA, 0)
            gx_ref[1, tpb] = b1 + lax.dot_general(
                xd_ref[tpb], wihT_ref[1], (((1,), (0,)), ((), ())),
                preferred_element_type=jnp.float32)

        def step(t, h, c, W, d):
            gates = gx_ref[d, t] + lax.dot_general(
                h, W, (((1,), (0,)), ((), ())),
                preferred_element_type=jnp.float32)
            a = jnp.tanh(gates)
            ig = 0.5 * a[:, :H] + 0.5
            fg = 0.5 * a[:, H:2 * H] + 0.5
            og = 0.5 * a[:, 3 * H:] + 0.5
            c_new = fg * c + ig * a[:, 2 * H:3 * H]
            h_new = og * jnp.tanh(c_new)
            mask = row_id < bs_ref[t]
            h = jnp.where(mask, h_new, h)
            c = jnp.where(mask, c_new, c)
            outd_ref[t, :, d * H:(d + 1) * H] = h
            return h, c

        def body(i, st):
            hf, cf, hb, cb = st
            t0 = 8 * i
            for k in range(8):
                t = t0 + k
                tb = T - 1 - t
                hf, cf = step(t, hf, cf, W0, 0)
                hb, cb = step(tb, hb, cb, W1, 1)
                prefetch(t)
            return (hf, cf, hb, cb)

        hf, cf, hb, cb = lax.fori_loop(
            0, T // 8, body, (h0_ref[0], c0_ref[0], h0_ref[1], c0_ref[1]))
        hout_ref[0] = hf
        cout_ref[0] = cf
        hout_ref[1] = hb
        cout_ref[1] = cb

        t = 0
        off = 0
        for bs, nsteps in runs:
            if bs == B:
                out_ref[off:off + nsteps * B, :] = (
                    outd_ref[t:t + nsteps].reshape(nsteps * B, 2 * H))
            else:
                for j in range(nsteps):
                    out_ref[off + j * bs:off + (j + 1) * bs, :] = (
                        outd_ref[t + j, 0:bs, :])
            t += nsteps
            off += bs * nsteps

    return kern


def kernel(input_data, batch_sizes, h0, c0, W_ih, W_hh, b_ih, b_hh):
    T = batch_sizes.shape[0]
    B = h0.shape[1]
    total, D = input_data.shape
    H = h0.shape[2]
    runs = _bs_runs(T, B)

    kern = _make_kernel(T, B, D, H, total, runs)
    out, h_out, c_out = pl.pallas_call(
        kern,
        out_shape=(
            jax.ShapeDtypeStruct((total, 2 * H), jnp.float32),
            jax.ShapeDtypeStruct((2, B, H), jnp.float32),
            jax.ShapeDtypeStruct((2, B, H), jnp.float32),
        ),
        in_specs=[
            pl.BlockSpec(memory_space=pltpu.VMEM),
            pl.BlockSpec(memory_space=pltpu.SMEM),
            pl.BlockSpec(memory_space=pltpu.VMEM),
            pl.BlockSpec(memory_space=pltpu.VMEM),
            pl.BlockSpec(memory_space=pltpu.VMEM),
            pl.BlockSpec(memory_space=pltpu.VMEM),
            pl.BlockSpec(memory_space=pltpu.VMEM),
            pl.BlockSpec(memory_space=pltpu.VMEM),
        ],
        scratch_shapes=[
            pltpu.VMEM((T, B, D), jnp.float32),
            pltpu.VMEM((2, T, B, 4 * H), jnp.float32),
            pltpu.VMEM((T, B, 2 * H), jnp.float32),
            pltpu.VMEM((2, D, 4 * H), jnp.float32),
            pltpu.VMEM((2, H, 4 * H), jnp.float32),
        ],
    )(
        input_data,
        batch_sizes,
        h0,
        c0,
        W_ih,
        W_hh,
        b_ih.reshape(2, 1, 4 * H),
        b_hh.reshape(2, 1, 4 * H),
    )
    return (out, h_out, c_out)


# speedup vs baseline: 1.1669x; 1.0237x over previous
import numpy as np
import jax
import jax.numpy as jnp
from jax import lax
from jax.experimental import pallas as pl
from jax.experimental.pallas import tpu as pltpu


def _bs_runs(T, B):
    lengths = np.array([T - 32 * i for i in range(B)], dtype=np.int64)
    bs_list = [int((lengths > t).sum()) for t in range(T)]
    runs = []
    for bs in bs_list:
        if runs and runs[-1][0] == bs:
            runs[-1][1] += 1
        else:
            runs.append([bs, 1])
    return [(int(b), int(n)) for b, n in runs]


def _make_kernel(T, B, D, H, total, runs):
    def kern(x_ref, bs_ref, h0_ref, c0_ref, wih_ref, whh_ref, bih_ref,
             bhh_ref, out_ref, hout_ref, cout_ref, xd_ref, gx_ref, outd_ref,
             wihT_ref, whhT_ref):
        col = lax.broadcasted_iota(jnp.int32, (1, 4 * H), 1)
        colscale = jnp.where((col >= 2 * H) & (col < 3 * H),
                             jnp.float32(1.0), jnp.float32(0.5))
        for d in range(2):
            wihT_ref[d] = lax.transpose(wih_ref[d], (1, 0)) * colscale
            whhT_ref[d] = lax.transpose(whh_ref[d], (1, 0)) * colscale
        b0 = (bih_ref[0] + bhh_ref[0]) * colscale
        b1 = (bih_ref[1] + bhh_ref[1]) * colscale
        b_col = (b0, b1)
        t = 0
        off = 0
        for bs, nsteps in runs:
            if bs == B:
                xd_ref[t:t + nsteps] = (
                    x_ref[off:off + nsteps * B, :].reshape(nsteps, B, D))
            else:
                for j in range(nsteps):
                    xd_ref[t + j, 0:bs, :] = (
                        x_ref[off + j * bs:off + (j + 1) * bs, :])
            t += nsteps
            off += bs * nsteps

        LA = 16
        for d, sl in ((0, slice(0, LA)), (1, slice(T - LA, T))):
            gxb = lax.dot_general(
                xd_ref[sl].reshape(LA * B, D), wihT_ref[d],
                (((1,), (0,)), ((), ())),
                preferred_element_type=jnp.float32)
            gx_ref[d, sl] = (gxb + b_col[d]).reshape(LA, B, 4 * H)

        row_id = lax.broadcasted_iota(jnp.int32, (B, 1), 0)
        W0 = whhT_ref[0]
        W1 = whhT_ref[1]

        def prefetch(t):
            tpf = jnp.minimum(t + LA, T - 1)
            gx_ref[0, tpf] = b0 + lax.dot_general(
                xd_ref[tpf], wihT_ref[0], (((1,), (0,)), ((), ())),
                preferred_element_type=jnp.float32)
            tpb = jnp.maximum(T - 1 - t - LA, 0)
            gx_ref[1, tpb] = b1 + lax.dot_general(
                xd_ref[tpb], wihT_ref[1], (((1,), (0,)), ((), ())),
                preferred_element_type=jnp.float32)

        def step(t, h, c, W, d):
            gates = gx_ref[d, t] + lax.dot_general(
                h, W, (((1,), (0,)), ((), ())),
                preferred_element_type=jnp.float32)
            a = jnp.tanh(gates)
            ig = 0.5 * a[:, :H] + 0.5
            fg = 0.5 * a[:, H:2 * H] + 0.5
            og = 0.5 * a[:, 3 * H:] + 0.5
            c_new = fg * c + ig * a[:, 2 * H:3 * H]
            h_new = og * jnp.tanh(c_new)
            mask = row_id < bs_ref[t]
            h = jnp.where(mask, h_new, h)
            c = jnp.where(mask, c_new, c)
            outd_ref[t, :, d * H:(d + 1) * H] = h
            return h, c

        def body(i, st):
            hf, cf, hb, cb = st
            t0 = 16 * i
            for k in range(16):
                t = t0 + k
                tb = T - 1 - t
                hf, cf = step(t, hf, cf, W0, 0)
                hb, cb = step(tb, hb, cb, W1, 1)
                prefetch(t)
            return (hf, cf, hb, cb)

        hf, cf, hb, cb = lax.fori_loop(
            0, T // 16, body, (h0_ref[0], c0_ref[0], h0_ref[1], c0_ref[1]))
        hout_ref[0] = hf
        cout_ref[0] = cf
        hout_ref[1] = hb
        cout_ref[1] = cb

        t = 0
        off = 0
        for bs, nsteps in runs:
            if bs == B:
                out_ref[off:off + nsteps * B, :] = (
                    outd_ref[t:t + nsteps].reshape(nsteps * B, 2 * H))
            else:
                for j in range(nsteps):
                    out_ref[off + j * bs:off + (j + 1) * bs, :] = (
                        outd_ref[t + j, 0:bs, :])
            t += nsteps
            off += bs * nsteps

    return kern


def kernel(input_data, batch_sizes, h0, c0, W_ih, W_hh, b_ih, b_hh):
    T = batch_sizes.shape[0]
    B = h0.shape[1]
    total, D = input_data.shape
    H = h0.shape[2]
    runs = _bs_runs(T, B)

    kern = _make_kernel(T, B, D, H, total, runs)
    out, h_out, c_out = pl.pallas_call(
        kern,
        out_shape=(
            jax.ShapeDtypeStruct((total, 2 * H), jnp.float32),
            jax.ShapeDtypeStruct((2, B, H), jnp.float32),
            jax.ShapeDtypeStruct((2, B, H), jnp.float32),
        ),
        in_specs=[
            pl.BlockSpec(memory_space=pltpu.VMEM),
            pl.BlockSpec(memory_space=pltpu.SMEM),
            pl.BlockSpec(memory_space=pltpu.VMEM),
            pl.BlockSpec(memory_space=pltpu.VMEM),
            pl.BlockSpec(memory_space=pltpu.VMEM),
            pl.BlockSpec(memory_space=pltpu.VMEM),
            pl.BlockSpec(memory_space=pltpu.VMEM),
            pl.BlockSpec(memory_space=pltpu.VMEM),
        ],
        scratch_shapes=[
            pltpu.VMEM((T, B, D), jnp.float32),
            pltpu.VMEM((2, T, B, 4 * H), jnp.float32),
            pltpu.VMEM((T, B, 2 * H), jnp.float32),
            pltpu.VMEM((2, D, 4 * H), jnp.float32),
            pltpu.VMEM((2, H, 4 * H), jnp.float32),
        ],
    )(
        input_data,
        batch_sizes,
        h0,
        c0,
        W_ih,
        W_hh,
        b_ih.reshape(2, 1, 4 * H),
        b_hh.reshape(2, 1, 4 * H),
    )
    return (out, h_out, c_out)


# speedup vs baseline: 1.1760x; 1.0078x over previous
import numpy as np
import jax
import jax.numpy as jnp
from jax import lax
from jax.experimental import pallas as pl
from jax.experimental.pallas import tpu as pltpu


def _bs_runs(T, B):
    lengths = np.array([T - 32 * i for i in range(B)], dtype=np.int64)
    bs_list = [int((lengths > t).sum()) for t in range(T)]
    runs = []
    for bs in bs_list:
        if runs and runs[-1][0] == bs:
            runs[-1][1] += 1
        else:
            runs.append([bs, 1])
    return [(int(b), int(n)) for b, n in runs]


def _make_kernel(T, B, D, H, total, runs):
    def kern(x_ref, bs_ref, h0_ref, c0_ref, wih_ref, whh_ref, bih_ref,
             bhh_ref, out_ref, hout_ref, cout_ref, xd_ref, gx_ref, outd_ref,
             wihT_ref, whhT_ref):
        col = lax.broadcasted_iota(jnp.int32, (1, 4 * H), 1)
        colscale = jnp.where((col >= 2 * H) & (col < 3 * H),
                             jnp.float32(1.0), jnp.float32(0.5))
        for d in range(2):
            wihT_ref[d] = lax.transpose(wih_ref[d], (1, 0)) * colscale
            whhT_ref[d] = lax.transpose(whh_ref[d], (1, 0)) * colscale
        b0 = (bih_ref[0] + bhh_ref[0]) * colscale
        b1 = (bih_ref[1] + bhh_ref[1]) * colscale
        b_col = (b0, b1)
        t = 0
        off = 0
        for bs, nsteps in runs:
            if bs == B:
                xd_ref[t:t + nsteps] = (
                    x_ref[off:off + nsteps * B, :].reshape(nsteps, B, D))
            else:
                for j in range(nsteps):
                    xd_ref[t + j, 0:bs, :] = (
                        x_ref[off + j * bs:off + (j + 1) * bs, :])
            t += nsteps
            off += bs * nsteps

        LA = 32
        for d, sl in ((0, slice(0, LA)), (1, slice(T - LA, T))):
            gxb = lax.dot_general(
                xd_ref[sl].reshape(LA * B, D), wihT_ref[d],
                (((1,), (0,)), ((), ())),
                preferred_element_type=jnp.float32)
            gx_ref[d, sl] = (gxb + b_col[d]).reshape(LA, B, 4 * H)

        row_id = lax.broadcasted_iota(jnp.int32, (B, 1), 0)
        W0 = whhT_ref[0]
        W1 = whhT_ref[1]

        def prefetch(t):
            tpf = jnp.minimum(t + LA, T - 1)
            gx_ref[0, tpf] = b0 + lax.dot_general(
                xd_ref[tpf], wihT_ref[0], (((1,), (0,)), ((), ())),
                preferred_element_type=jnp.float32)
            tpb = jnp.maximum(T - 1 - t - LA, 0)
            gx_ref[1, tpb] = b1 + lax.dot_general(
                xd_ref[tpb], wihT_ref[1], (((1,), (0,)), ((), ())),
                preferred_element_type=jnp.float32)

        def step(t, h, c, W, d):
            gates = gx_ref[d, t] + lax.dot_general(
                h, W, (((1,), (0,)), ((), ())),
                preferred_element_type=jnp.float32)
            a = jnp.tanh(gates)
            ig = 0.5 * a[:, :H] + 0.5
            fg = 0.5 * a[:, H:2 * H] + 0.5
            og = 0.5 * a[:, 3 * H:] + 0.5
            c_new = fg * c + ig * a[:, 2 * H:3 * H]
            h_new = og * jnp.tanh(c_new)
            mask = row_id < bs_ref[t]
            h = jnp.where(mask, h_new, h)
            c = jnp.where(mask, c_new, c)
            outd_ref[t, :, d * H:(d + 1) * H] = h
            return h, c

        def body(i, st):
            hf, cf, hb, cb = st
            t0 = 32 * i
            for k in range(32):
                t = t0 + k
                tb = T - 1 - t
                hf, cf = step(t, hf, cf, W0, 0)
                hb, cb = step(tb, hb, cb, W1, 1)
                prefetch(t)
            return (hf, cf, hb, cb)

        hf, cf, hb, cb = lax.fori_loop(
            0, T // 32, body, (h0_ref[0], c0_ref[0], h0_ref[1], c0_ref[1]))
        hout_ref[0] = hf
        cout_ref[0] = cf
        hout_ref[1] = hb
        cout_ref[1] = cb

        t = 0
        off = 0
        for bs, nsteps in runs:
            if bs == B:
                out_ref[off:off + nsteps * B, :] = (
                    outd_ref[t:t + nsteps].reshape(nsteps * B, 2 * H))
            else:
                for j in range(nsteps):
                    out_ref[off + j * bs:off + (j + 1) * bs, :] = (
                        outd_ref[t + j, 0:bs, :])
            t += nsteps
            off += bs * nsteps

    return kern


def kernel(input_data, batch_sizes, h0, c0, W_ih, W_hh, b_ih, b_hh):
    T = batch_sizes.shape[0]
    B = h0.shape[1]
    total, D = input_data.shape
    H = h0.shape[2]
    runs = _bs_runs(T, B)

    kern = _make_kernel(T, B, D, H, total, runs)
    out, h_out, c_out = pl.pallas_call(
        kern,
        out_shape=(
            jax.ShapeDtypeStruct((total, 2 * H), jnp.float32),
            jax.ShapeDtypeStruct((2, B, H), jnp.float32),
            jax.ShapeDtypeStruct((2, B, H), jnp.float32),
        ),
        in_specs=[
            pl.BlockSpec(memory_space=pltpu.VMEM),
            pl.BlockSpec(memory_space=pltpu.SMEM),
            pl.BlockSpec(memory_space=pltpu.VMEM),
            pl.BlockSpec(memory_space=pltpu.VMEM),
            pl.BlockSpec(memory_space=pltpu.VMEM),
            pl.BlockSpec(memory_space=pltpu.VMEM),
            pl.BlockSpec(memory_space=pltpu.VMEM),
            pl.BlockSpec(memory_space=pltpu.VMEM),
        ],
        scratch_shapes=[
            pltpu.VMEM((T, B, D), jnp.float32),
            pltpu.VMEM((2, T, B, 4 * H), jnp.float32),
            pltpu.VMEM((T, B, 2 * H), jnp.float32),
            pltpu.VMEM((2, D, 4 * H), jnp.float32),
            pltpu.VMEM((2, H, 4 * H), jnp.float32),
        ],
    )(
        input_data,
        batch_sizes,
        h0,
        c0,
        W_ih,
        W_hh,
        b_ih.reshape(2, 1, 4 * H),
        b_hh.reshape(2, 1, 4 * H),
    )
    return (out, h_out, c_out)


# speedup vs baseline: 1.1766x; 1.0005x over previous
import numpy as np
import jax
import jax.numpy as jnp
from jax import lax
from jax.experimental import pallas as pl
from jax.experimental.pallas import tpu as pltpu


def _bs_runs(T, B):
    lengths = np.array([T - 32 * i for i in range(B)], dtype=np.int64)
    bs_list = [int((lengths > t).sum()) for t in range(T)]
    runs = []
    for bs in bs_list:
        if runs and runs[-1][0] == bs:
            runs[-1][1] += 1
        else:
            runs.append([bs, 1])
    return [(int(b), int(n)) for b, n in runs]


def _make_kernel(T, B, D, H, total, runs):
    def kern(x_ref, bs_ref, h0_ref, c0_ref, wih_ref, whh_ref, bih_ref,
             bhh_ref, out_ref, hout_ref, cout_ref, xd_ref, outd_ref,
             wcat_ref):
        col = lax.broadcasted_iota(jnp.int32, (1, 4 * H), 1)
        colscale = jnp.where((col >= 2 * H) & (col < 3 * H),
                             jnp.float32(1.0), jnp.float32(0.5))
        for d in range(2):
            wcat_ref[d, 0:D] = lax.transpose(wih_ref[d], (1, 0)) * colscale
            wcat_ref[d, D:D + H] = lax.transpose(whh_ref[d], (1, 0)) * colscale
        b0 = (bih_ref[0] + bhh_ref[0]) * colscale
        b1 = (bih_ref[1] + bhh_ref[1]) * colscale
        b_col = (b0, b1)
        t = 0
        off = 0
        for bs, nsteps in runs:
            if bs == B:
                xd_ref[t:t + nsteps] = (
                    x_ref[off:off + nsteps * B, :].reshape(nsteps, B, D))
            else:
                for j in range(nsteps):
                    xd_ref[t + j, 0:bs, :] = (
                        x_ref[off + j * bs:off + (j + 1) * bs, :])
            t += nsteps
            off += bs * nsteps

        row_id = lax.broadcasted_iota(jnp.int32, (B, 1), 0)
        W0 = wcat_ref[0]
        W1 = wcat_ref[1]

        def step(t, h, c, W, d):
            xh = jnp.concatenate([xd_ref[t], h], 1)
            gates = b_col[d] + lax.dot_general(
                xh, W, (((1,), (0,)), ((), ())),
                preferred_element_type=jnp.float32)
            a = jnp.tanh(gates)
            ig = 0.5 * a[:, :H] + 0.5
            fg = 0.5 * a[:, H:2 * H] + 0.5
            og = 0.5 * a[:, 3 * H:] + 0.5
            c_new = fg * c + ig * a[:, 2 * H:3 * H]
            h_new = og * jnp.tanh(c_new)
            mask = row_id < bs_ref[t]
            h = jnp.where(mask, h_new, h)
            c = jnp.where(mask, c_new, c)
            outd_ref[t, :, d * H:(d + 1) * H] = h
            return h, c

        def body(i, st):
            hf, cf, hb, cb = st
            t0 = 32 * i
            for k in range(32):
                t = t0 + k
                tb = T - 1 - t
                hf, cf = step(t, hf, cf, W0, 0)
                hb, cb = step(tb, hb, cb, W1, 1)
            return (hf, cf, hb, cb)

        hf, cf, hb, cb = lax.fori_loop(
            0, T // 32, body, (h0_ref[0], c0_ref[0], h0_ref[1], c0_ref[1]))
        hout_ref[0] = hf
        cout_ref[0] = cf
        hout_ref[1] = hb
        cout_ref[1] = cb

        t = 0
        off = 0
        for bs, nsteps in runs:
            if bs == B:
                out_ref[off:off + nsteps * B, :] = (
                    outd_ref[t:t + nsteps].reshape(nsteps * B, 2 * H))
            else:
                for j in range(nsteps):
                    out_ref[off + j * bs:off + (j + 1) * bs, :] = (
                        outd_ref[t + j, 0:bs, :])
            t += nsteps
            off += bs * nsteps

    return kern


def kernel(input_data, batch_sizes, h0, c0, W_ih, W_hh, b_ih, b_hh):
    T = batch_sizes.shape[0]
    B = h0.shape[1]
    total, D = input_data.shape
    H = h0.shape[2]
    runs = _bs_runs(T, B)

    kern = _make_kernel(T, B, D, H, total, runs)
    out, h_out, c_out = pl.pallas_call(
        kern,
        out_shape=(
            jax.ShapeDtypeStruct((total, 2 * H), jnp.float32),
            jax.ShapeDtypeStruct((2, B, H), jnp.float32),
            jax.ShapeDtypeStruct((2, B, H), jnp.float32),
        ),
        in_specs=[
            pl.BlockSpec(memory_space=pltpu.VMEM),
            pl.BlockSpec(memory_space=pltpu.SMEM),
            pl.BlockSpec(memory_space=pltpu.VMEM),
            pl.BlockSpec(memory_space=pltpu.VMEM),
            pl.BlockSpec(memory_space=pltpu.VMEM),
            pl.BlockSpec(memory_space=pltpu.VMEM),
            pl.BlockSpec(memory_space=pltpu.VMEM),
            pl.BlockSpec(memory_space=pltpu.VMEM),
        ],
        scratch_shapes=[
            pltpu.VMEM((T, B, D), jnp.float32),
            pltpu.VMEM((T, B, 2 * H), jnp.float32),
            pltpu.VMEM((2, D + H, 4 * H), jnp.float32),
        ],
    )(
        input_data,
        batch_sizes,
        h0,
        c0,
        W_ih,
        W_hh,
        b_ih.reshape(2, 1, 4 * H),
        b_hh.reshape(2, 1, 4 * H),
    )
    return (out, h_out, c_out)
